# baseline (speedup 1.0000x reference)
import functools

import jax
import jax.numpy as jnp
from jax import lax
from jax.experimental import pallas as pl
from jax.experimental.pallas import tpu as pltpu
from jax.experimental.pallas import tpu_sc as plsc

N = 10000
E = 320000
D = 128
H = 128
O = 2

NC = 2
NS = 16
NW = NC * NS
CH = 125
CPW = 80
EPW = CH * CPW

_mesh = plsc.VectorSubcoreMesh(
    core_axis_name="c", subcore_axis_name="s", num_cores=NC, num_subcores=NS
)


def _segsum_body(with_counts, table, ei2, *rest):
    if with_counts:
        (out, cnt_out, rows_a, rows_b, src_a, src_b, src_c2, src_d,
         dst_a, dst_b, dst_c2, dst_d, cnt_v, acc,
         gsem_a, gsem_b, ssem_a, ssem_b, ssem_c, ssem_d,
         dsem_a, dsem_b, dsem_c, dsem_d) = rest
    else:
        (out, rows_a, rows_b, src_a, src_b, src_c2, src_d,
         dst_a, dst_b, dst_c2, dst_d, acc,
         gsem_a, gsem_b, ssem_a, ssem_b, ssem_c, ssem_d,
         dsem_a, dsem_b, dsem_c, dsem_d) = rest
        cnt_v = None
    cid = lax.axis_index("c")
    sid = lax.axis_index("s")
    wid = sid * NC + cid

    zero16f = jnp.zeros((16,), jnp.float32)

    @pl.loop(0, CH * D // 16)
    def _zr(i):
        rows_a[(i * 16) // D, pl.ds((i * 16) % D, 16)] = zero16f

    @pl.loop(0, N // (NS * CH))
    def _za(j):
        pltpu.sync_copy(rows_a, acc.at[pl.ds((sid * (N // (NS * CH)) + j) * CH, CH)])

    if with_counts:
        zero16 = jnp.zeros((16,), jnp.float32)

        @pl.loop(0, N // 16)
        def _z(i):
            cnt_v[pl.ds(i * 16, 16)] = zero16

        one16 = jnp.ones((16,), jnp.float32)
        tail_mask = lax.iota(jnp.int32, 16) >= 3

    plsc.subcore_barrier()

    base = wid * CPW

    def _prefetch_idx(c, sbuf, dbuf, ssem, dsem):
        pltpu.async_copy(ei2.at[base + c], sbuf, ssem)
        pltpu.async_copy(ei2.at[NW * CPW + base + c], dbuf, dsem)

    def _launch_gather(c, sbuf, rows_buf, ssem, gsem):
        pltpu.make_async_copy(ei2.at[base + c], sbuf, ssem).wait()
        pltpu.async_copy(table.at[sbuf], rows_buf, gsem)

    def _consume(c, sbuf, dbuf, rows_buf, ssem, dsem, gsem):
        pltpu.make_async_copy(ei2.at[NW * CPW + base + c], dbuf, dsem).wait()
        pltpu.make_async_copy(table.at[sbuf], rows_buf, gsem).wait()
        pltpu.sync_copy(rows_buf, acc.at[dbuf], add=True)
        if with_counts:
            for j in range(7):
                d16 = dbuf[pl.ds(j * 16, 16)]
                plsc.addupdate_scatter(cnt_v, [d16], one16)
            d16 = dbuf[pl.ds(CH - 16, 16)]
            plsc.addupdate_scatter(cnt_v, [d16], one16, mask=tail_mask)

    srcs = (src_a, src_b, src_c2, src_d)
    dsts = (dst_a, dst_b, dst_c2, dst_d)
    ssems = (ssem_a, ssem_b, ssem_c, ssem_d)
    dsems = (dsem_a, dsem_b, dsem_c, dsem_d)
    rows = (rows_a, rows_b)
    gsems = (gsem_a, gsem_b)

    for k in range(4):
        _prefetch_idx(k, srcs[k], dsts[k], ssems[k], dsems[k])
    _launch_gather(0, srcs[0], rows[0], ssems[0], gsems[0])

    @pl.loop(0, CPW // 4)
    def _chunk(c4):
        c0 = c4 * 4
        for ph in range(4):
            c = c0 + ph
            k = ph
            kn = (ph + 1) % 4
            r = ph % 2
            rn = (ph + 1) % 2

            @pl.when(c + 1 < CPW)
            def _():
                _launch_gather(c + 1, srcs[kn], rows[rn], ssems[kn], gsems[rn])

            _consume(c, srcs[k], dsts[k], rows[r], ssems[k], dsems[k], gsems[r])

            @pl.when(c + 4 < CPW)
            def _():
                _prefetch_idx(c + 4, srcs[k], dsts[k], ssems[k], dsems[k])

    if with_counts:
        pltpu.sync_copy(cnt_v, cnt_out.at[wid])

    plsc.subcore_barrier()

    @pl.when(sid == 0)
    def _():
        pltpu.sync_copy(acc, out.at[cid])


def _make_segsum(with_counts):
    parts = jax.ShapeDtypeStruct((NC, N, D), jnp.float32)
    cnts = jax.ShapeDtypeStruct((NW, N), jnp.float32)
    scratch = [
        pltpu.VMEM((CH, D), jnp.float32),
        pltpu.VMEM((CH, D), jnp.float32),
    ] + [pltpu.VMEM((CH,), jnp.int32)] * 8
    if with_counts:
        scratch.append(pltpu.VMEM((N,), jnp.float32))
    scratch += [pltpu.VMEM_SHARED((N, D), jnp.float32)]
    scratch += [pltpu.SemaphoreType.DMA] * 10
    return pl.kernel(
        functools.partial(_segsum_body, with_counts),
        out_type=(parts, cnts) if with_counts else parts,
        mesh=_mesh,
        compiler_params=pltpu.CompilerParams(needs_layout_passes=False),
        scratch_types=scratch,
    )


_segsum_cnt = _make_segsum(True)
_segsum = _make_segsum(False)


def _edge_body(ab, ei2e, s0_out, s1_out, ab_v, src_v, dst_v, o0_v, o1_v,
               sem_ab, sem_s, sem_d):
    cid = lax.axis_index("c")
    sid = lax.axis_index("s")
    wid = sid * NC + cid

    cp_ab = pltpu.async_copy(ab, ab_v, sem_ab)
    cp_s = pltpu.async_copy(ei2e.at[pl.ds(wid * CPW, CPW)], src_v, sem_s)
    cp_d = pltpu.async_copy(
        ei2e.at[pl.ds(NW * CPW + wid * CPW, CPW)], dst_v, sem_d)
    cp_ab.wait()
    cp_s.wait()
    cp_d.wait()

    offs = tuple(16 * j for j in range(7)) + (CH - 16,)

    @pl.loop(0, CPW)
    def _row(c):
        rbase = c * CH
        for off in offs:
            s16 = src_v[c, pl.ds(off, 16)] * 4
            d16 = dst_v[c, pl.ds(off, 16)] * 4
            a0 = plsc.load_gather(ab_v, [s16])
            a1 = plsc.load_gather(ab_v, [s16 + 1])
            b0 = plsc.load_gather(ab_v, [d16 + 2])
            b1 = plsc.load_gather(ab_v, [d16 + 3])
            o0_v[pl.ds(rbase + off, 16)] = a0 + b0
            o1_v[pl.ds(rbase + off, 16)] = a1 + b1

    pltpu.sync_copy(o0_v, s0_out.at[pl.ds(wid * EPW, EPW)])
    pltpu.sync_copy(o1_v, s1_out.at[pl.ds(wid * EPW, EPW)])


_edge = pl.kernel(
    _edge_body,
    out_type=(
        jax.ShapeDtypeStruct((E,), jnp.float32),
        jax.ShapeDtypeStruct((E,), jnp.float32),
    ),
    mesh=_mesh,
    compiler_params=pltpu.CompilerParams(needs_layout_passes=False),
    scratch_types=[
        pltpu.VMEM((N * 4,), jnp.float32),
        pltpu.VMEM((CPW, CH), jnp.int32),
        pltpu.VMEM((CPW, CH), jnp.int32),
        pltpu.VMEM((EPW,), jnp.float32),
        pltpu.VMEM((EPW,), jnp.float32),
        pltpu.SemaphoreType.DMA,
        pltpu.SemaphoreType.DMA,
        pltpu.SemaphoreType.DMA,
    ],
)


_RB = 2000

_CONTRACT0 = (((0,), (0,)), ((), ()))


def _cnt_reduce_body(cp, out):
    ones = jnp.ones((NW, 1), jnp.float32)
    cnt = lax.dot_general(cp[...], ones, _CONTRACT0,
                          preferred_element_type=jnp.float32)
    out[...] = jnp.maximum(cnt, 1.0)


def _cnt_reduce(cp):
    return pl.pallas_call(
        _cnt_reduce_body,
        out_shape=jax.ShapeDtypeStruct((N, 1), jnp.float32),
    )(cp)


def _layer_tc_body(cp, p, xin, wl, br, wr, out):
    s = p[0] + p[1]
    mean = s / cp[...]
    h = jnp.dot(mean, wl[...], preferred_element_type=jnp.float32)
    h = h + br[...] + jnp.dot(xin[...], wr[...], preferred_element_type=jnp.float32)
    out[...] = jnp.maximum(h, 0.0)


def _layer_tc(cp, p, xin, wl, br, wr):
    blk = lambda i: (i, 0)
    cnt_blk = pl.BlockSpec((_RB, 1), blk)
    w_blk = pl.BlockSpec((D, H), lambda i: (0, 0))
    return pl.pallas_call(
        _layer_tc_body,
        grid=(N // _RB,),
        in_specs=[
            cnt_blk,
            pl.BlockSpec((NC, _RB, D), lambda i: (0, i, 0)),
            pl.BlockSpec((_RB, D), blk),
            w_blk,
            pl.BlockSpec((1, H), lambda i: (0, 0)),
            w_blk,
        ],
        out_specs=pl.BlockSpec((_RB, H), blk),
        out_shape=jax.ShapeDtypeStruct((N, H), jnp.float32),
    )(cp, p, xin, wl, br, wr)


def _final_tc_body(cp, p, hin, wl, br, wr, wt, wb, bf, out):
    s = p[0] + p[1]
    mean = s / cp[...]
    h = jnp.dot(mean, wl[...], preferred_element_type=jnp.float32)
    h = h + br[...] + jnp.dot(hin[...], wr[...], preferred_element_type=jnp.float32)
    h = jnp.maximum(h, 0.0)
    a = jnp.dot(h, wt[...], preferred_element_type=jnp.float32) + bf[...]
    b = jnp.dot(h, wb[...], preferred_element_type=jnp.float32)
    out[...] = jnp.concatenate([a, b], axis=1)


def _final_tc(cp, p, hin, wl, br, wr, wt, wb, bf):
    blk = lambda i: (i, 0)
    cnt_blk = pl.BlockSpec((_RB, 1), blk)
    w_blk = pl.BlockSpec((D, H), lambda i: (0, 0))
    return pl.pallas_call(
        _final_tc_body,
        grid=(N // _RB,),
        in_specs=[
            cnt_blk,
            pl.BlockSpec((NC, _RB, D), lambda i: (0, i, 0)),
            pl.BlockSpec((_RB, D), blk),
            w_blk,
            pl.BlockSpec((1, H), lambda i: (0, 0)),
            w_blk,
            pl.BlockSpec((H, O), lambda i: (0, 0)),
            pl.BlockSpec((H, O), lambda i: (0, 0)),
            pl.BlockSpec((1, O), lambda i: (0, 0)),
        ],
        out_specs=pl.BlockSpec((_RB, 4), blk),
        out_shape=jax.ShapeDtypeStruct((N, 4), jnp.float32),
    )(cp, p, hin, wl, br, wr, wt, wb, bf)


def _lsm_body(s0, s1, o0, o1):
    a = s0[...]
    b = s1[...]
    m = jnp.maximum(a, b)
    lse = m + jnp.log(jnp.exp(a - m) + jnp.exp(b - m))
    o0[...] = a - lse
    o1[...] = b - lse


def _lsm(s0, s1):
    return pl.pallas_call(
        _lsm_body,
        out_shape=(
            jax.ShapeDtypeStruct(s0.shape, jnp.float32),
            jax.ShapeDtypeStruct(s0.shape, jnp.float32),
        ),
    )(s0, s1)


def kernel(x, edge_index, W1_l, b1, W1_r, W2_l, b2, W2_r, Wfc, bfc):
    ei2 = edge_index.reshape(2 * NW * CPW, CH)

    p, cnt_parts = _segsum_cnt(x, ei2)
    cnt_col = _cnt_reduce(cnt_parts)
    h1 = _layer_tc(cnt_col, p, x, W1_l, b1.reshape(1, H), W1_r)
    p2 = _segsum(h1, ei2)
    ab = _final_tc(
        cnt_col, p2, h1, W2_l, b2.reshape(1, H), W2_r,
        Wfc[:H], Wfc[H:], bfc.reshape(1, O),
    )
    s0, s1 = _edge(ab.reshape(-1), ei2)
    o0, o1 = _lsm(s0.reshape(E // 128, 128), s1.reshape(E // 128, 128))
    return jnp.stack([o0.reshape(-1), o1.reshape(-1)], axis=-1)


# speedup vs baseline: 1.0150x; 1.0150x over previous
import functools

import jax
import jax.numpy as jnp
from jax import lax
from jax.experimental import pallas as pl
from jax.experimental.pallas import tpu as pltpu
from jax.experimental.pallas import tpu_sc as plsc

N = 10000
E = 320000
D = 128
H = 128
O = 2

NC = 2
NS = 16
NW = NC * NS
CH = 125
CPW = 80
EPW = CH * CPW

_mesh = plsc.VectorSubcoreMesh(
    core_axis_name="c", subcore_axis_name="s", num_cores=NC, num_subcores=NS
)


def _segsum_body(with_counts, table, ei2, *rest):
    if with_counts:
        (out, cnt_out, rows_a, rows_b, src_a, src_b, src_c2, src_d,
         dst_a, dst_b, dst_c2, dst_d, cnt_v, acc,
         gsem_a, gsem_b, ssem_a, ssem_b, ssem_c, ssem_d,
         dsem_a, dsem_b, dsem_c, dsem_d, scsem) = rest
    else:
        (out, rows_a, rows_b, src_a, src_b, src_c2, src_d,
         dst_a, dst_b, dst_c2, dst_d, acc,
         gsem_a, gsem_b, ssem_a, ssem_b, ssem_c, ssem_d,
         dsem_a, dsem_b, dsem_c, dsem_d, scsem) = rest
        cnt_v = None
    cid = lax.axis_index("c")
    sid = lax.axis_index("s")
    wid = sid * NC + cid

    zero16f = jnp.zeros((16,), jnp.float32)

    @pl.loop(0, CH * D // 16)
    def _zr(i):
        rows_a[(i * 16) // D, pl.ds((i * 16) % D, 16)] = zero16f

    @pl.loop(0, N // (NS * CH))
    def _za(j):
        pltpu.sync_copy(rows_a, acc.at[pl.ds((sid * (N // (NS * CH)) + j) * CH, CH)])

    if with_counts:
        zero16 = jnp.zeros((16,), jnp.float32)

        @pl.loop(0, N // 16)
        def _z(i):
            cnt_v[pl.ds(i * 16, 16)] = zero16

        one16 = jnp.ones((16,), jnp.float32)
        tail_mask = lax.iota(jnp.int32, 16) >= 3

    plsc.subcore_barrier()

    base = wid * CPW

    def _prefetch_idx(c, sbuf, dbuf, ssem, dsem):
        pltpu.async_copy(ei2.at[base + c], sbuf, ssem)
        pltpu.async_copy(ei2.at[NW * CPW + base + c], dbuf, dsem)

    def _launch_gather(c, sbuf, rows_buf, ssem, gsem):
        pltpu.make_async_copy(ei2.at[base + c], sbuf, ssem).wait()
        pltpu.async_copy(table.at[sbuf], rows_buf, gsem)

    def _consume(c, sbuf, dbuf, rows_buf, ssem, dsem, gsem):
        pltpu.make_async_copy(ei2.at[NW * CPW + base + c], dbuf, dsem).wait()
        pltpu.make_async_copy(table.at[sbuf], rows_buf, gsem).wait()
        cp = pltpu.async_copy(rows_buf, acc.at[dbuf], scsem, add=True)
        if with_counts:
            for j in range(7):
                d16 = dbuf[pl.ds(j * 16, 16)]
                plsc.addupdate_scatter(cnt_v, [d16], one16)
            d16 = dbuf[pl.ds(CH - 16, 16)]
            plsc.addupdate_scatter(cnt_v, [d16], one16, mask=tail_mask)
        cp.wait()

    srcs = (src_a, src_b, src_c2, src_d)
    dsts = (dst_a, dst_b, dst_c2, dst_d)
    ssems = (ssem_a, ssem_b, ssem_c, ssem_d)
    dsems = (dsem_a, dsem_b, dsem_c, dsem_d)
    rows = (rows_a, rows_b)
    gsems = (gsem_a, gsem_b)

    for k in range(4):
        _prefetch_idx(k, srcs[k], dsts[k], ssems[k], dsems[k])
    _launch_gather(0, srcs[0], rows[0], ssems[0], gsems[0])

    @pl.loop(0, CPW // 4)
    def _chunk(c4):
        c0 = c4 * 4
        for ph in range(4):
            c = c0 + ph
            k = ph
            kn = (ph + 1) % 4
            r = ph % 2
            rn = (ph + 1) % 2

            @pl.when(c + 1 < CPW)
            def _():
                _launch_gather(c + 1, srcs[kn], rows[rn], ssems[kn], gsems[rn])

            _consume(c, srcs[k], dsts[k], rows[r], ssems[k], dsems[k], gsems[r])

            @pl.when(c + 4 < CPW)
            def _():
                _prefetch_idx(c + 4, srcs[k], dsts[k], ssems[k], dsems[k])

    if with_counts:
        pltpu.sync_copy(cnt_v, cnt_out.at[wid])

    plsc.subcore_barrier()

    @pl.when(sid == 0)
    def _():
        pltpu.sync_copy(acc, out.at[cid])


def _make_segsum(with_counts):
    parts = jax.ShapeDtypeStruct((NC, N, D), jnp.float32)
    cnts = jax.ShapeDtypeStruct((NW, N), jnp.float32)
    scratch = [
        pltpu.VMEM((CH, D), jnp.float32),
        pltpu.VMEM((CH, D), jnp.float32),
    ] + [pltpu.VMEM((CH,), jnp.int32)] * 8
    if with_counts:
        scratch.append(pltpu.VMEM((N,), jnp.float32))
    scratch += [pltpu.VMEM_SHARED((N, D), jnp.float32)]
    scratch += [pltpu.SemaphoreType.DMA] * 11
    return pl.kernel(
        functools.partial(_segsum_body, with_counts),
        out_type=(parts, cnts) if with_counts else parts,
        mesh=_mesh,
        compiler_params=pltpu.CompilerParams(needs_layout_passes=False),
        scratch_types=scratch,
    )


_segsum_cnt = _make_segsum(True)
_segsum = _make_segsum(False)


def _edge_body(ab, eiw, s0_out, s1_out, ab_v, src_v, dst_v, o0_v, o1_v,
               sem_ab, sem_s, sem_d):
    cid = lax.axis_index("c")
    sid = lax.axis_index("s")
    wid = sid * NC + cid

    cp_ab = pltpu.async_copy(ab, ab_v, sem_ab)
    cp_s = pltpu.async_copy(eiw.at[wid], src_v, sem_s)
    cp_d = pltpu.async_copy(eiw.at[NW + wid], dst_v, sem_d)
    cp_ab.wait()
    cp_s.wait()
    cp_d.wait()

    @pl.loop(0, EPW // 16)
    def _grp(i):
        s16 = src_v[pl.ds(i * 16, 16)] * 4
        d16 = dst_v[pl.ds(i * 16, 16)] * 4
        a0 = plsc.load_gather(ab_v, [s16])
        a1 = plsc.load_gather(ab_v, [s16 + 1])
        b0 = plsc.load_gather(ab_v, [d16 + 2])
        b1 = plsc.load_gather(ab_v, [d16 + 3])
        o0_v[pl.ds(i * 16, 16)] = a0 + b0
        o1_v[pl.ds(i * 16, 16)] = a1 + b1

    pltpu.sync_copy(o0_v, s0_out.at[pl.ds(wid * EPW, EPW)])
    pltpu.sync_copy(o1_v, s1_out.at[pl.ds(wid * EPW, EPW)])


_edge = pl.kernel(
    _edge_body,
    out_type=(
        jax.ShapeDtypeStruct((E,), jnp.float32),
        jax.ShapeDtypeStruct((E,), jnp.float32),
    ),
    mesh=_mesh,
    compiler_params=pltpu.CompilerParams(needs_layout_passes=False),
    scratch_types=[
        pltpu.VMEM((N * 4,), jnp.float32),
        pltpu.VMEM((EPW,), jnp.int32),
        pltpu.VMEM((EPW,), jnp.int32),
        pltpu.VMEM((EPW,), jnp.float32),
        pltpu.VMEM((EPW,), jnp.float32),
        pltpu.SemaphoreType.DMA,
        pltpu.SemaphoreType.DMA,
        pltpu.SemaphoreType.DMA,
    ],
)


_RB = 2000

_CONTRACT0 = (((0,), (0,)), ((), ()))


def _cnt_reduce_body(cp, out):
    ones = jnp.ones((NW, 1), jnp.float32)
    cnt = lax.dot_general(cp[...], ones, _CONTRACT0,
                          preferred_element_type=jnp.float32)
    out[...] = jnp.maximum(cnt, 1.0)


def _cnt_reduce(cp):
    return pl.pallas_call(
        _cnt_reduce_body,
        out_shape=jax.ShapeDtypeStruct((N, 1), jnp.float32),
    )(cp)


def _layer_tc_body(cp, p, xin, wl, br, wr, out):
    s = p[0] + p[1]
    mean = s / cp[...]
    h = jnp.dot(mean, wl[...], preferred_element_type=jnp.float32)
    h = h + br[...] + jnp.dot(xin[...], wr[...], preferred_element_type=jnp.float32)
    out[...] = jnp.maximum(h, 0.0)


def _layer_tc(cp, p, xin, wl, br, wr):
    blk = lambda i: (i, 0)
    cnt_blk = pl.BlockSpec((_RB, 1), blk)
    w_blk = pl.BlockSpec((D, H), lambda i: (0, 0))
    return pl.pallas_call(
        _layer_tc_body,
        grid=(N // _RB,),
        in_specs=[
            cnt_blk,
            pl.BlockSpec((NC, _RB, D), lambda i: (0, i, 0)),
            pl.BlockSpec((_RB, D), blk),
            w_blk,
            pl.BlockSpec((1, H), lambda i: (0, 0)),
            w_blk,
        ],
        out_specs=pl.BlockSpec((_RB, H), blk),
        out_shape=jax.ShapeDtypeStruct((N, H), jnp.float32),
    )(cp, p, xin, wl, br, wr)


def _final_tc_body(cp, p, hin, wl, br, wr, wt, wb, bf, out):
    s = p[0] + p[1]
    mean = s / cp[...]
    h = jnp.dot(mean, wl[...], preferred_element_type=jnp.float32)
    h = h + br[...] + jnp.dot(hin[...], wr[...], preferred_element_type=jnp.float32)
    h = jnp.maximum(h, 0.0)
    a = jnp.dot(h, wt[...], preferred_element_type=jnp.float32) + bf[...]
    b = jnp.dot(h, wb[...], preferred_element_type=jnp.float32)
    out[...] = jnp.concatenate([a, b], axis=1)


def _final_tc(cp, p, hin, wl, br, wr, wt, wb, bf):
    blk = lambda i: (i, 0)
    cnt_blk = pl.BlockSpec((_RB, 1), blk)
    w_blk = pl.BlockSpec((D, H), lambda i: (0, 0))
    return pl.pallas_call(
        _final_tc_body,
        grid=(N // _RB,),
        in_specs=[
            cnt_blk,
            pl.BlockSpec((NC, _RB, D), lambda i: (0, i, 0)),
            pl.BlockSpec((_RB, D), blk),
            w_blk,
            pl.BlockSpec((1, H), lambda i: (0, 0)),
            w_blk,
            pl.BlockSpec((H, O), lambda i: (0, 0)),
            pl.BlockSpec((H, O), lambda i: (0, 0)),
            pl.BlockSpec((1, O), lambda i: (0, 0)),
        ],
        out_specs=pl.BlockSpec((_RB, 4), blk),
        out_shape=jax.ShapeDtypeStruct((N, 4), jnp.float32),
    )(cp, p, hin, wl, br, wr, wt, wb, bf)


def _lsm_body(s0, s1, o0, o1):
    a = s0[...]
    b = s1[...]
    m = jnp.maximum(a, b)
    lse = m + jnp.log(jnp.exp(a - m) + jnp.exp(b - m))
    o0[...] = a - lse
    o1[...] = b - lse


def _lsm(s0, s1):
    return pl.pallas_call(
        _lsm_body,
        out_shape=(
            jax.ShapeDtypeStruct(s0.shape, jnp.float32),
            jax.ShapeDtypeStruct(s0.shape, jnp.float32),
        ),
    )(s0, s1)


def kernel(x, edge_index, W1_l, b1, W1_r, W2_l, b2, W2_r, Wfc, bfc):
    ei2 = edge_index.reshape(2 * NW * CPW, CH)
    eiw = edge_index.reshape(2 * NW, EPW)

    p, cnt_parts = _segsum_cnt(x, ei2)
    cnt_col = _cnt_reduce(cnt_parts)
    h1 = _layer_tc(cnt_col, p, x, W1_l, b1.reshape(1, H), W1_r)
    p2 = _segsum(h1, ei2)
    ab = _final_tc(
        cnt_col, p2, h1, W2_l, b2.reshape(1, H), W2_r,
        Wfc[:H], Wfc[H:], bfc.reshape(1, O),
    )
    s0, s1 = _edge(ab.reshape(-1), eiw)
    o0, o1 = _lsm(s0.reshape(E // 128, 128), s1.reshape(E // 128, 128))
    return jnp.stack([o0.reshape(-1), o1.reshape(-1)], axis=-1)
